# baseline (speedup 1.0000x reference)
import functools

import jax
import jax.numpy as jnp
from jax import lax
from jax.experimental import pallas as pl
from jax.experimental.pallas import tpu as pltpu
from jax.experimental.pallas import tpu_sc as plsc

_N = 16384
_C = 10000
_NC = 2
_NS = 16
_NW = _NC * _NS
_PW = _N // _NW
_CHUNK = 128
_NCH = _PW // _CHUNK
_L = 16


def _body(prob_hbm, tgt_hbm, rew_hbm, out_hbm, idx_v, val_v, rew_v, acc_v, sem):
    cid = lax.axis_index("c")
    sid = lax.axis_index("s")
    wid = sid * _NC + cid
    base = wid * _PW

    for j in range(_NCH):
        pltpu.sync_copy(tgt_hbm.at[pl.ds(base + j * _CHUNK, _CHUNK)], idx_v.at[j])
    pltpu.sync_copy(rew_hbm.at[pl.ds(base, _PW)], rew_v)

    lane = lax.iota(jnp.int32, _L)
    for j in range(_NCH):
        for g in range(_CHUNK // _L):
            row0 = base + j * _CHUNK + g * _L
            t = idx_v[j, pl.ds(g * _L, _L)]
            idx_v[j, pl.ds(g * _L, _L)] = t + (row0 + lane) * _C

    copies = [
        pltpu.async_copy(
            prob_hbm.at[idx_v.at[j]], val_v.at[pl.ds(j * _CHUNK, _CHUNK)], sem
        )
        for j in range(_NCH)
    ]
    for c in copies:
        c.wait()

    acc = jnp.zeros((_L,), jnp.float32)
    for t in range(_PW // _L):
        acc = acc + val_v[pl.ds(t * _L, _L)] * rew_v[pl.ds(t * _L, _L)]
    acc_v[...] = acc * (-1.0 / _N)
    pltpu.sync_copy(acc_v, out_hbm.at[wid])


_gather_loss = functools.partial(
    pl.kernel,
    out_type=jax.ShapeDtypeStruct((_NW, _L), jnp.float32),
    mesh=plsc.VectorSubcoreMesh(core_axis_name="c", subcore_axis_name="s"),
    scratch_types=[
        pltpu.VMEM((_NCH, _CHUNK), jnp.int32),
        pltpu.VMEM((_PW,), jnp.float32),
        pltpu.VMEM((_PW,), jnp.float32),
        pltpu.VMEM((_L,), jnp.float32),
        pltpu.SemaphoreType.DMA,
    ],
)(_body)


def kernel(prob, targets, reward):
    part = _gather_loss(prob.reshape(-1), targets, reward)
    return jnp.sum(part)


# speedup vs baseline: 43.4960x; 43.4960x over previous
import functools

import jax
import jax.numpy as jnp
from jax import lax
from jax.experimental import pallas as pl
from jax.experimental.pallas import tpu as pltpu
from jax.experimental.pallas import tpu_sc as plsc

_N = 16384
_C = 10000
_NC = 2
_NS = 16
_NW = _NC * _NS
_PW = _N // _NW
_CHUNK = 128
_NCH = _PW // _CHUNK
_L = 16


def _body(prob_hbm, tgt_hbm, rew_hbm, out_hbm, idx_v, val_v, rew_v, acc_v, sem):
    cid = lax.axis_index("c")
    sid = lax.axis_index("s")
    wid = sid * _NC + cid
    base = wid * _PW

    for j in range(_NCH):
        pltpu.sync_copy(tgt_hbm.at[pl.ds(base + j * _CHUNK, _CHUNK)], idx_v.at[j])
    pltpu.sync_copy(rew_hbm.at[pl.ds(base, _PW)], rew_v)

    lane = lax.iota(jnp.int32, _L)
    for j in range(_NCH):
        for g in range(_CHUNK // _L):
            row0 = base + j * _CHUNK + g * _L
            r = row0 + lane
            c = idx_v[j, pl.ds(g * _L, _L)]
            k = (
                lax.shift_left(lax.shift_right_logical(c, 3), 17)
                + lax.shift_left(lax.shift_right_logical(r, 7), 10)
                + lax.shift_left(lax.bitwise_and(c, 7), 7)
                + lax.bitwise_and(r, 127)
            )
            idx_v[j, pl.ds(g * _L, _L)] = k

    copies = [
        pltpu.async_copy(
            prob_hbm.at[idx_v.at[j]], val_v.at[pl.ds(j * _CHUNK, _CHUNK)], sem
        )
        for j in range(_NCH)
    ]
    for c in copies:
        c.wait()

    acc = jnp.zeros((_L,), jnp.float32)
    for t in range(_PW // _L):
        acc = acc + val_v[pl.ds(t * _L, _L)] * rew_v[pl.ds(t * _L, _L)]
    acc_v[...] = acc * (-1.0 / _N)
    pltpu.sync_copy(acc_v, out_hbm.at[wid])


_gather_loss = functools.partial(
    pl.kernel,
    out_type=jax.ShapeDtypeStruct((_NW, _L), jnp.float32),
    mesh=plsc.VectorSubcoreMesh(core_axis_name="c", subcore_axis_name="s"),
    scratch_types=[
        pltpu.VMEM((_NCH, _CHUNK), jnp.int32),
        pltpu.VMEM((_PW,), jnp.float32),
        pltpu.VMEM((_PW,), jnp.float32),
        pltpu.VMEM((_L,), jnp.float32),
        pltpu.SemaphoreType.DMA,
    ],
)(_body)


def kernel(prob, targets, reward):
    pflat = prob.reshape(128, 128, 1250, 8).transpose(2, 0, 3, 1).reshape(-1)
    part = _gather_loss(pflat, targets, reward)
    return jnp.sum(part)


# speedup vs baseline: 47.2013x; 1.0852x over previous
import functools

import jax
import jax.numpy as jnp
from jax import lax
from jax.experimental import pallas as pl
from jax.experimental.pallas import tpu as pltpu
from jax.experimental.pallas import tpu_sc as plsc

_N = 16384
_C = 10000
_NC = 2
_NS = 16
_NW = _NC * _NS
_PW = _N // _NW
_CHUNK = 128
_NCH = _PW // _CHUNK
_L = 16


def _body(prob_hbm, tgt_hbm, rew_hbm, out_hbm,
          tgt_v, idx_v, val_v, rew_v, acc_v, tsem, rsem, gsem):
    cid = lax.axis_index("c")
    sid = lax.axis_index("s")
    wid = sid * _NC + cid
    base = wid * _PW

    tcopy = pltpu.async_copy(tgt_hbm.at[pl.ds(base, _PW)], tgt_v, tsem)
    rcopy = pltpu.async_copy(rew_hbm.at[pl.ds(base, _PW)], rew_v, rsem)
    tcopy.wait()

    lane = lax.iota(jnp.int32, _L)
    gathers = []
    for j in range(_NCH):
        for g in range(_CHUNK // _L):
            x = base + j * _CHUNK + g * _L
            rpart = (
                lax.shift_left(lax.shift_right_logical(x, 7), 10)
                + lax.bitwise_and(x, 127)
            )
            c = tgt_v[pl.ds(j * _CHUNK + g * _L, _L)]
            k = (
                lax.shift_left(lax.bitwise_and(c, jnp.int32(-8)), 14)
                + lax.shift_left(lax.bitwise_and(c, 7), 7)
                + (rpart + lane)
            )
            idx_v[j, pl.ds(g * _L, _L)] = k
        gathers.append(
            pltpu.async_copy(
                prob_hbm.at[idx_v.at[j]],
                val_v.at[pl.ds(j * _CHUNK, _CHUNK)],
                gsem,
            )
        )

    rcopy.wait()
    for cpy in gathers:
        cpy.wait()

    acc = jnp.zeros((_L,), jnp.float32)
    for t in range(_PW // _L):
        acc = acc + val_v[pl.ds(t * _L, _L)] * rew_v[pl.ds(t * _L, _L)]
    acc_v[...] = acc * (-1.0 / _N)
    pltpu.sync_copy(acc_v, out_hbm.at[wid])


_gather_loss = functools.partial(
    pl.kernel,
    out_type=jax.ShapeDtypeStruct((_NW, _L), jnp.float32),
    mesh=plsc.VectorSubcoreMesh(core_axis_name="c", subcore_axis_name="s"),
    scratch_types=[
        pltpu.VMEM((_PW,), jnp.int32),
        pltpu.VMEM((_NCH, _CHUNK), jnp.int32),
        pltpu.VMEM((_PW,), jnp.float32),
        pltpu.VMEM((_PW,), jnp.float32),
        pltpu.VMEM((_L,), jnp.float32),
        pltpu.SemaphoreType.DMA,
        pltpu.SemaphoreType.DMA,
        pltpu.SemaphoreType.DMA,
    ],
)(_body)


def kernel(prob, targets, reward):
    pflat = prob.reshape(128, 128, 1250, 8).transpose(2, 0, 3, 1).reshape(-1)
    part = _gather_loss(pflat, targets, reward)
    return jnp.sum(part)


# speedup vs baseline: 47.5149x; 1.0066x over previous
import functools

import jax
import jax.numpy as jnp
from jax import lax
from jax.experimental import pallas as pl
from jax.experimental.pallas import tpu as pltpu
from jax.experimental.pallas import tpu_sc as plsc

_N = 16384
_C = 10000
_NC = 2
_NS = 16
_NW = _NC * _NS
_PW = _N // _NW
_CHUNK = 128
_NCH = _PW // _CHUNK
_L = 16


def _body(prob_hbm, tgt_hbm, rew_hbm, out_hbm,
          tgt_v, idx_v, val_v, rew_v, acc_v,
          t0sem, t1sem, rsem, g0sem, g1sem, g2sem, g3sem):
    cid = lax.axis_index("c")
    sid = lax.axis_index("s")
    wid = sid * _NC + cid
    base = wid * _PW

    half = _PW // 2
    tcopy0 = pltpu.async_copy(tgt_hbm.at[pl.ds(base, half)],
                              tgt_v.at[pl.ds(0, half)], t0sem)
    tcopy1 = pltpu.async_copy(tgt_hbm.at[pl.ds(base + half, half)],
                              tgt_v.at[pl.ds(half, half)], t1sem)
    rcopy = pltpu.async_copy(rew_hbm.at[pl.ds(base, _PW)], rew_v, rsem)

    lane = lax.iota(jnp.int32, _L)
    gsems = [g0sem, g1sem, g2sem, g3sem]
    gathers = []
    tcopy0.wait()
    for j in range(_NCH):
        if j == _NCH // 2:
            tcopy1.wait()
        for g in range(_CHUNK // _L):
            x = base + j * _CHUNK + g * _L
            rpart = (
                lax.shift_left(lax.shift_right_logical(x, 7), 10)
                + lax.bitwise_and(x, 127)
            )
            c = tgt_v[pl.ds(j * _CHUNK + g * _L, _L)]
            k = (
                lax.shift_left(lax.bitwise_and(c, jnp.int32(-8)), 14)
                + lax.shift_left(lax.bitwise_and(c, 7), 7)
                + (rpart + lane)
            )
            idx_v[j, pl.ds(g * _L, _L)] = k
        gathers.append(
            pltpu.async_copy(
                prob_hbm.at[idx_v.at[j]],
                val_v.at[pl.ds(j * _CHUNK, _CHUNK)],
                gsems[j],
            )
        )

    rcopy.wait()
    acc0 = jnp.zeros((_L,), jnp.float32)
    acc1 = jnp.zeros((_L,), jnp.float32)
    for j in range(_NCH):
        gathers[j].wait()
        for g in range(0, _CHUNK // _L, 2):
            t = j * (_CHUNK // _L) + g
            acc0 = acc0 + val_v[pl.ds(t * _L, _L)] * rew_v[pl.ds(t * _L, _L)]
            acc1 = acc1 + val_v[pl.ds((t + 1) * _L, _L)] * rew_v[pl.ds((t + 1) * _L, _L)]
    acc_v[...] = (acc0 + acc1) * (-1.0 / _N)
    pltpu.sync_copy(acc_v, out_hbm.at[wid])


_gather_loss = functools.partial(
    pl.kernel,
    out_type=jax.ShapeDtypeStruct((_NW, _L), jnp.float32),
    mesh=plsc.VectorSubcoreMesh(core_axis_name="c", subcore_axis_name="s"),
    scratch_types=[
        pltpu.VMEM((_PW,), jnp.int32),
        pltpu.VMEM((_NCH, _CHUNK), jnp.int32),
        pltpu.VMEM((_PW,), jnp.float32),
        pltpu.VMEM((_PW,), jnp.float32),
        pltpu.VMEM((_L,), jnp.float32),
        pltpu.SemaphoreType.DMA,
        pltpu.SemaphoreType.DMA,
        pltpu.SemaphoreType.DMA,
        pltpu.SemaphoreType.DMA,
        pltpu.SemaphoreType.DMA,
        pltpu.SemaphoreType.DMA,
        pltpu.SemaphoreType.DMA,
    ],
)(_body)


def kernel(prob, targets, reward):
    pflat = prob.reshape(128, 128, 1250, 8).transpose(2, 0, 3, 1).reshape(-1)
    part = _gather_loss(pflat, targets, reward)
    return jnp.sum(part)


# speedup vs baseline: 47.7616x; 1.0052x over previous
import functools

import jax
import jax.numpy as jnp
from jax import lax
from jax.experimental import pallas as pl
from jax.experimental.pallas import tpu as pltpu
from jax.experimental.pallas import tpu_sc as plsc

_N = 16384
_C = 10000
_NC = 2
_NS = 16
_NW = _NC * _NS
_PW = _N // _NW
_CHUNK = 128
_NCH = _PW // _CHUNK
_L = 16


def _body(prob_hbm, tgt_hbm, rew_hbm, out_hbm,
          tgt_v, idx_v, val_v, rew_v, acc_v,
          t0sem, t1sem, rsem, g0sem, g1sem, g2sem, g3sem):
    cid = lax.axis_index("c")
    sid = lax.axis_index("s")
    wid = sid * _NC + cid
    base = wid * _PW

    half = _PW // 2
    tcopy0 = pltpu.async_copy(tgt_hbm.at[pl.ds(base, half)],
                              tgt_v.at[pl.ds(0, half)], t0sem)
    tcopy1 = pltpu.async_copy(tgt_hbm.at[pl.ds(base + half, half)],
                              tgt_v.at[pl.ds(half, half)], t1sem)
    rcopy = pltpu.async_copy(rew_hbm.at[pl.ds(base, _PW)], rew_v, rsem)

    lane = lax.iota(jnp.int32, _L)
    gsems = [g0sem, g1sem, g2sem, g3sem]
    gathers = []
    wid4096 = lax.shift_left(wid, 12)
    tcopy0.wait()
    for j in range(_NCH):
        if j == _NCH // 2:
            tcopy1.wait()
        for g in range(_CHUNK // _L):
            off = j * _CHUNK + g * _L
            rconst = ((off >> 7) << 10) + (off & 127)
            c = tgt_v[pl.ds(off, _L)]
            k = (
                lax.shift_left(lax.bitwise_and(c, jnp.int32(-8)), 14)
                + lax.shift_left(lax.bitwise_and(c, 7), 7)
                + (wid4096 + (rconst + lane))
            )
            idx_v[j, pl.ds(g * _L, _L)] = k
        gathers.append(
            pltpu.async_copy(
                prob_hbm.at[idx_v.at[j]],
                val_v.at[pl.ds(j * _CHUNK, _CHUNK)],
                gsems[j],
            )
        )

    rcopy.wait()
    acc0 = jnp.zeros((_L,), jnp.float32)
    acc1 = jnp.zeros((_L,), jnp.float32)
    for j in range(_NCH):
        gathers[j].wait()
        for g in range(0, _CHUNK // _L, 2):
            t = j * (_CHUNK // _L) + g
            acc0 = acc0 + val_v[pl.ds(t * _L, _L)] * rew_v[pl.ds(t * _L, _L)]
            acc1 = acc1 + val_v[pl.ds((t + 1) * _L, _L)] * rew_v[pl.ds((t + 1) * _L, _L)]
    acc_v[...] = (acc0 + acc1) * (-1.0 / _N)
    pltpu.sync_copy(acc_v, out_hbm.at[wid])


_gather_loss = functools.partial(
    pl.kernel,
    out_type=jax.ShapeDtypeStruct((_NW, _L), jnp.float32),
    mesh=plsc.VectorSubcoreMesh(core_axis_name="c", subcore_axis_name="s"),
    scratch_types=[
        pltpu.VMEM((_PW,), jnp.int32),
        pltpu.VMEM((_NCH, _CHUNK), jnp.int32),
        pltpu.VMEM((_PW,), jnp.float32),
        pltpu.VMEM((_PW,), jnp.float32),
        pltpu.VMEM((_L,), jnp.float32),
        pltpu.SemaphoreType.DMA,
        pltpu.SemaphoreType.DMA,
        pltpu.SemaphoreType.DMA,
        pltpu.SemaphoreType.DMA,
        pltpu.SemaphoreType.DMA,
        pltpu.SemaphoreType.DMA,
        pltpu.SemaphoreType.DMA,
    ],
)(_body)


def kernel(prob, targets, reward):
    pflat = prob.reshape(128, 128, 1250, 8).transpose(2, 0, 3, 1).reshape(-1)
    part = _gather_loss(pflat, targets, reward)
    return jnp.sum(part)


# speedup vs baseline: 47.9347x; 1.0036x over previous
import functools

import jax
import jax.numpy as jnp
from jax import lax
from jax.experimental import pallas as pl
from jax.experimental.pallas import tpu as pltpu
from jax.experimental.pallas import tpu_sc as plsc

_N = 16384
_C = 10000
_NC = 1
_NS = 16
_NW = _NC * _NS
_PW = _N // _NW
_CHUNK = 128
_NCH = _PW // _CHUNK
_L = 16


def _body(prob_hbm, tgt_hbm, rew_hbm, out_hbm,
          tgt_v, idx_v, val_v, rew_v, acc_v, t0sem, t1sem, rsem, gsem):
    cid = lax.axis_index("c")
    sid = lax.axis_index("s")
    wid = sid * _NC + cid
    base = wid * _PW

    half = _PW // 2
    tcopy0 = pltpu.async_copy(tgt_hbm.at[pl.ds(base, half)],
                              tgt_v.at[pl.ds(0, half)], t0sem)
    tcopy1 = pltpu.async_copy(tgt_hbm.at[pl.ds(base + half, half)],
                              tgt_v.at[pl.ds(half, half)], t1sem)
    rcopy = pltpu.async_copy(rew_hbm.at[pl.ds(base, _PW)], rew_v, rsem)

    lane = lax.iota(jnp.int32, _L)
    gathers = []
    widr = lax.shift_left(wid, 13)
    tcopy0.wait()
    for j in range(_NCH):
        if j == _NCH // 2:
            tcopy1.wait()
        for g in range(_CHUNK // _L):
            off = j * _CHUNK + g * _L
            rconst = ((off >> 7) << 10) + (off & 127)
            c = tgt_v[pl.ds(off, _L)]
            k = (
                lax.shift_left(lax.bitwise_and(c, jnp.int32(-8)), 14)
                + lax.shift_left(lax.bitwise_and(c, 7), 7)
                + (widr + (rconst + lane))
            )
            idx_v[j, pl.ds(g * _L, _L)] = k
        gathers.append(
            pltpu.async_copy(
                prob_hbm.at[idx_v.at[j]],
                val_v.at[pl.ds(j * _CHUNK, _CHUNK)],
                gsem,
            )
        )

    rcopy.wait()
    for cpy in gathers:
        cpy.wait()

    acc0 = jnp.zeros((_L,), jnp.float32)
    acc1 = jnp.zeros((_L,), jnp.float32)
    for t in range(0, _PW // _L, 2):
        acc0 = acc0 + val_v[pl.ds(t * _L, _L)] * rew_v[pl.ds(t * _L, _L)]
        acc1 = acc1 + val_v[pl.ds((t + 1) * _L, _L)] * rew_v[pl.ds((t + 1) * _L, _L)]
    acc_v[...] = (acc0 + acc1) * (-1.0 / _N)
    pltpu.sync_copy(acc_v, out_hbm.at[wid])


_gather_loss = functools.partial(
    pl.kernel,
    out_type=jax.ShapeDtypeStruct((_NW, _L), jnp.float32),
    mesh=plsc.VectorSubcoreMesh(core_axis_name="c", subcore_axis_name="s",
                                num_cores=1),
    scratch_types=[
        pltpu.VMEM((_PW,), jnp.int32),
        pltpu.VMEM((_NCH, _CHUNK), jnp.int32),
        pltpu.VMEM((_PW,), jnp.float32),
        pltpu.VMEM((_PW,), jnp.float32),
        pltpu.VMEM((_L,), jnp.float32),
        pltpu.SemaphoreType.DMA,
        pltpu.SemaphoreType.DMA,
        pltpu.SemaphoreType.DMA,
        pltpu.SemaphoreType.DMA,
    ],
)(_body)


def kernel(prob, targets, reward):
    pflat = prob.reshape(128, 128, 1250, 8).transpose(2, 0, 3, 1).reshape(-1)
    part = _gather_loss(pflat, targets, reward)
    return jnp.sum(part)
